# Initial kernel scaffold; baseline (speedup 1.0000x reference)
import jax
import jax.numpy as jnp
from jax.experimental import pallas as pl


def kernel(ent_table, rel_table, W, b, node_idx, edge_index, edge_type, batch_idx):
    raise NotImplementedError("write your pallas kernel here")


# speedup vs baseline: 3.2095x; 3.2095x over previous
import functools

import jax
import jax.numpy as jnp
from jax.experimental import pallas as pl
from jax.experimental.pallas import tpu as pltpu
from jax.experimental.pallas import tpu_sc as plsc

N = 10000
E = 320000
D = 128
H = 4
R = 64
B = 1024

NC = 2
NS = 16
NW = NC * NS

EPW = E // NW
KE = 80
NCH_E = EPW // KE

KN = 80
NCH_N = N // KN
BPT = B // NS

_mesh = plsc.VectorSubcoreMesh(
    core_axis_name="c", subcore_axis_name="s", num_cores=NC, num_subcores=NS
)


def _worker_id():
    return jax.lax.axis_index("s") * NC + jax.lax.axis_index("c")


def _zero_buf(zbuf, rows):
    def body(i, _):
        for j in range(D // 16):
            zbuf[i, pl.ds(j * 16, 16)] = jnp.zeros((16,), jnp.float32)
        return 0
    jax.lax.fori_loop(0, rows, body, 0)


@functools.partial(
    pl.kernel,
    out_type=jax.ShapeDtypeStruct((N, D), jnp.float32),
    mesh=_mesh,
    scratch_types=[
        pltpu.VMEM((KN,), jnp.int32),
        pltpu.VMEM((KN, D), jnp.float32),
    ],
)
def _gather_x(ent_hbm, nidx_hbm, x_hbm, idx_v, rows_v):
    w = _worker_id()

    def chunk(k, _):
        ci = w + NW * k

        @pl.when(ci < NCH_N)
        def _():
            r0 = ci * KN
            pltpu.sync_copy(nidx_hbm.at[pl.ds(r0, KN)], idx_v)
            pltpu.sync_copy(ent_hbm.at[idx_v], rows_v)
            pltpu.sync_copy(rows_v, x_hbm.at[pl.ds(r0, KN)])
        return 0

    jax.lax.fori_loop(0, (NCH_N + NW - 1) // NW, chunk, 0)


def _make_edge_pass(hp_count):
    @functools.partial(
        pl.kernel,
        out_type=jax.ShapeDtypeStruct((hp_count, NC, N, D), jnp.float32),
        mesh=_mesh,
        scratch_types=[
            pltpu.VMEM((KE,), jnp.int32),
            pltpu.VMEM((KE,), jnp.int32),
            pltpu.VMEM((KE,), jnp.int32),
            pltpu.VMEM((KE, D), jnp.float32),
            pltpu.VMEM((KE, D), jnp.float32),
            pltpu.VMEM((KN, D), jnp.float32),
            pltpu.VMEM_SHARED((N, D), jnp.float32),
            pltpu.VMEM_SHARED((R, D), jnp.float32),
        ],
    )
    def edge_pass(*refs):
        h_hbms = refs[:hp_count]
        src_hbm, dst_hbm, typ_hbm, rel_hbm, out_hbm = refs[hp_count:hp_count + 5]
        (src_v, dst_v, typ_v, rows_v, rel_v, zbuf, agg_sh, rel_sh) = refs[hp_count + 5:]

        cid = jax.lax.axis_index("c")
        sid = jax.lax.axis_index("s")
        w = _worker_id()

        _zero_buf(zbuf, KN)

        @pl.when(sid == 0)
        def _():
            pltpu.sync_copy(rel_hbm, rel_sh)

        for hp in range(hp_count):
            def zero_chunk(k, _):
                ci = sid + NS * k

                @pl.when(ci < NCH_N)
                def _():
                    pltpu.sync_copy(zbuf, agg_sh.at[pl.ds(ci * KN, KN)])
                return 0
            jax.lax.fori_loop(0, (NCH_N + NS - 1) // NS, zero_chunk, 0)
            plsc.subcore_barrier()

            def chunk(ci, _):
                base = w * EPW + ci * KE
                pltpu.sync_copy(src_hbm.at[pl.ds(base, KE)], src_v)
                pltpu.sync_copy(typ_hbm.at[pl.ds(base, KE)], typ_v)
                pltpu.sync_copy(dst_hbm.at[pl.ds(base, KE)], dst_v)
                pltpu.sync_copy(h_hbms[hp].at[src_v], rows_v)
                pltpu.sync_copy(rel_sh.at[typ_v], rel_v)

                def mul(e, _):
                    for j in range(D // 16):
                        sl = pl.ds(j * 16, 16)
                        rows_v[e, sl] = rows_v[e, sl] * rel_v[e, sl]
                    return 0
                jax.lax.fori_loop(0, KE, mul, 0)

                pltpu.sync_copy(rows_v, agg_sh.at[dst_v], add=True)
                return 0
            jax.lax.fori_loop(0, NCH_E, chunk, 0)
            plsc.subcore_barrier()

            def out_chunk(k, _):
                ci = sid + NS * k

                @pl.when(ci < NCH_N)
                def _():
                    r0 = ci * KN
                    pltpu.sync_copy(
                        agg_sh.at[pl.ds(r0, KN)], out_hbm.at[hp, cid, pl.ds(r0, KN)]
                    )
                return 0
            jax.lax.fori_loop(0, (NCH_N + NS - 1) // NS, out_chunk, 0)
            plsc.subcore_barrier()

    return edge_pass


_edge_pass_1 = _make_edge_pass(1)
_edge_pass_4 = _make_edge_pass(4)


BN = 1000


def _linear_body(hp_in, p_ref, w_ref, b_ref, o_ref):
    for hp in range(H):
        a = p_ref[min(hp, hp_in - 1), 0] + p_ref[min(hp, hp_in - 1), 1]
        y = jnp.dot(a, w_ref[hp], preferred_element_type=jnp.float32)
        y = y + b_ref[hp][None, :]
        o_ref[hp] = jnp.maximum(y, 0.0)


def _linear(p, w_l, b_l):
    hp_in = p.shape[0]
    return pl.pallas_call(
        functools.partial(_linear_body, hp_in),
        grid=(N // BN,),
        in_specs=[
            pl.BlockSpec((hp_in, NC, BN, D), lambda i: (0, 0, i, 0)),
            pl.BlockSpec((H, D, D), lambda i: (0, 0, 0)),
            pl.BlockSpec((H, D), lambda i: (0, 0)),
        ],
        out_specs=pl.BlockSpec((H, BN, D), lambda i: (0, i, 0)),
        out_shape=jax.ShapeDtypeStruct((H, N, D), jnp.float32),
    )(p, w_l, b_l)


@functools.partial(
    pl.kernel,
    out_type=jax.ShapeDtypeStruct((H, NC, B, D), jnp.float32),
    mesh=_mesh,
    scratch_types=[
        pltpu.VMEM((KN,), jnp.int32),
        pltpu.VMEM((KN, D), jnp.float32),
        pltpu.VMEM((BPT, D), jnp.float32),
        pltpu.VMEM_SHARED((B, D), jnp.float32),
    ],
)
def _readout(h0, h1, h2, h3, bidx_hbm, out_hbm, idx_v, rows_v, zbuf, acc_sh):
    h_hbms = (h0, h1, h2, h3)
    cid = jax.lax.axis_index("c")
    sid = jax.lax.axis_index("s")
    w = _worker_id()

    _zero_buf(zbuf, BPT)

    for hp in range(H):
        pltpu.sync_copy(zbuf, acc_sh.at[pl.ds(sid * BPT, BPT)])
        plsc.subcore_barrier()

        def chunk(k, _):
            ci = w + NW * k

            @pl.when(ci < NCH_N)
            def _():
                r0 = ci * KN
                pltpu.sync_copy(h_hbms[hp].at[pl.ds(r0, KN)], rows_v)
                pltpu.sync_copy(bidx_hbm.at[pl.ds(r0, KN)], idx_v)
                pltpu.sync_copy(rows_v, acc_sh.at[idx_v], add=True)
            return 0
        jax.lax.fori_loop(0, (NCH_N + NW - 1) // NW, chunk, 0)
        plsc.subcore_barrier()

        pltpu.sync_copy(
            acc_sh.at[pl.ds(sid * BPT, BPT)],
            out_hbm.at[hp, cid, pl.ds(sid * BPT, BPT)],
        )
        plsc.subcore_barrier()


def kernel(ent_table, rel_table, W, b, node_idx, edge_index, edge_type, batch_idx):
    node_idx = node_idx.astype(jnp.int32)
    src = edge_index[0].astype(jnp.int32)
    dst = edge_index[1].astype(jnp.int32)
    edge_type = edge_type.astype(jnp.int32)
    batch_idx = batch_idx.astype(jnp.int32)

    x = _gather_x(ent_table, node_idx)

    p0 = _edge_pass_1(x, src, dst, edge_type, rel_table)
    h1 = _linear(p0, W[:, 0], b[:, 0])

    p1 = _edge_pass_4(h1[0], h1[1], h1[2], h1[3], src, dst, edge_type, rel_table)
    h2 = _linear(p1, W[:, 1], b[:, 1])

    ro = _readout(h2[0], h2[1], h2[2], h2[3], batch_idx)
    out = ro.sum(axis=1)
    return jnp.transpose(out, (1, 0, 2)).reshape(B, H * D)
